# baseline (speedup 1.0000x reference)
import functools

import jax
import jax.numpy as jnp
from jax import lax
from jax.experimental import pallas as pl
from jax.experimental.pallas import tpu as pltpu
from jax.experimental.pallas import tpu_sc as plsc

VOCAB = 1000000
DIM = 32
BATCH = 16384

_NUM_CORES = 2
_NUM_SUBCORES = 16
_NW = _NUM_CORES * _NUM_SUBCORES
_TCOLS = 245
_WIN = _TCOLS * 128
_CW = 512
_NCHUNK = 64
_MAX_S = 999552
_NVEC = BATCH // 16


def _build_gather():
    mesh = plsc.VectorSubcoreMesh(core_axis_name="c", subcore_axis_name="s")

    @functools.partial(
        pl.kernel,
        mesh=mesh,
        out_type=jax.ShapeDtypeStruct((BATCH, 128), jnp.float32),
        scratch_types=[
            pltpu.VMEM((BATCH,), jnp.int32),
            pltpu.VMEM((BATCH,), jnp.int32),
            pltpu.VMEM((BATCH,), jnp.int32),
            pltpu.VMEM((DIM, _CW), jnp.float32),
            pltpu.VMEM((DIM, _CW), jnp.float32),
            pltpu.VMEM((DIM, _CW), jnp.float32),
            pltpu.VMEM((DIM, _CW), jnp.float32),
            pltpu.VMEM((16, 128), jnp.float32),
            pltpu.VMEM((16, 128), jnp.float32),
            pltpu.VMEM((16,), jnp.int32),
            pltpu.VMEM((16,), jnp.int32),
            pltpu.SemaphoreType.DMA,
            pltpu.SemaphoreType.DMA,
            pltpu.SemaphoreType.DMA,
            pltpu.SemaphoreType.DMA,
            pltpu.SemaphoreType.DMA,
            pltpu.SemaphoreType.DMA,
        ],
        compiler_params=pltpu.CompilerParams(needs_layout_passes=False),
    )
    def gather_kernel(tbl, idx_hbm, out_hbm, idx_v, dense, chits,
                      ch_a, ch_b, ch_c, ch_d, rb_a, rb_b, bv_a, bv_b,
                      sa, sb, sc, sd, wa, wb):
        w = lax.axis_index("s") * _NUM_CORES + lax.axis_index("c")
        lo = w * _WIN
        hi = jnp.minimum(lo + _WIN, VOCAB)
        iota = lax.iota(jnp.int32, 16)

        def chunk_start(k):
            return jnp.minimum(lo + k * _CW, _MAX_S)

        def enq_chunk(buf, k, sem):
            coff = pl.multiple_of(chunk_start(k), 128)
            pltpu.async_copy(tbl.at[:, pl.ds(coff, _CW)], buf, sem)

        def drain_chunk(buf, sem):
            pltpu.make_async_copy(tbl.at[:, pl.ds(0, _CW)], buf, sem).wait()

        pltpu.sync_copy(idx_hbm, idx_v)
        enq_chunk(ch_a, 0, sa)
        enq_chunk(ch_b, 1, sb)
        enq_chunk(ch_c, 2, sc)
        enq_chunk(ch_d, 3, sd)

        def scan_body(g, cur):
            iv = idx_v[pl.ds(pl.multiple_of(g * 16, 16), 16)]
            m = (iv >= lo) & (iv < hi)
            mi = m.astype(jnp.int32)
            cs = plsc.cumsum(mi)
            packed = (iv - lo) * BATCH + g * 16 + iota
            plsc.store_scatter(dense, [cur + cs - mi], packed, mask=m)
            return cur + cs[15]

        m_total = lax.fori_loop(0, _NVEC, scan_body, 0)
        n_dvec = (m_total + 15) >> 4

        def do_group(h0, m_k, s_rel, buf, rb, bv, sem, flag):
            guard = h0 < m_k

            @pl.when(guard)
            def _():
                hv = chits[pl.ds(pl.multiple_of(h0, 16), 16)]
                valid = (h0 + iota) < m_k
                hv2 = jnp.where(valid, hv, jnp.broadcast_to(hv[0], (16,)))
                jc = (hv2 >> 14) - s_rel
                b = hv2 & (BATCH - 1)

                @pl.when(flag > 0)
                def _():
                    pltpu.make_async_copy(rb, out_hbm.at[bv], sem).wait()

                for f in range(DIM):
                    fv = jnp.broadcast_to(jnp.int32(f), (16,))
                    vals = plsc.load_gather(buf, [fv, jc])
                    plsc.store_scatter(rb, [iota, fv], vals)
                bv[...] = b
                pltpu.async_copy(rb, out_hbm.at[bv], sem)

            return jnp.where(guard, jnp.int32(1), flag)

        def process_chunk(k, buf, sem, fa, fb):
            drain_chunk(buf, sem)
            s_rel = chunk_start(k) - lo

            def cc_body(v, cur):
                dv = dense[pl.ds(pl.multiple_of(v * 16, 16), 16)]
                rel = dv >> 14
                mm = ((v * 16 + iota) < m_total) \
                    & (rel >= k * _CW) & (rel < (k + 1) * _CW)
                mi = mm.astype(jnp.int32)
                cs = plsc.cumsum(mi)
                plsc.store_scatter(chits, [cur + cs - mi], dv, mask=mm)
                return cur + cs[15]

            m_k = lax.fori_loop(0, n_dvec, cc_body, 0)

            def ex_body(t, fs):
                f1 = do_group(t * 32, m_k, s_rel, buf, rb_a, bv_a, wa, fs[0])
                f2 = do_group(t * 32 + 16, m_k, s_rel, buf, rb_b, bv_b, wb,
                              fs[1])
                return (f1, f2)

            return lax.fori_loop(0, (m_k + 31) >> 5, ex_body, (fa, fb))

        def chunk_quad(j, fs):
            k = j * 4
            fa, fb = fs

            for off, buf, sem in ((0, ch_a, sa), (1, ch_b, sb),
                                  (2, ch_c, sc), (3, ch_d, sd)):
                fa, fb = process_chunk(k + off, buf, sem, fa, fb)

                @pl.when(k + off + 4 < _NCHUNK)
                def _(buf=buf, sem=sem, off=off):
                    enq_chunk(buf, k + off + 4, sem)

            return (fa, fb)

        fa, fb = lax.fori_loop(0, _NCHUNK // 4, chunk_quad,
                               (jnp.int32(0), jnp.int32(0)))

        @pl.when(fa > 0)
        def _():
            pltpu.make_async_copy(rb_a, out_hbm.at[bv_a], wa).wait()

        @pl.when(fb > 0)
        def _():
            pltpu.make_async_copy(rb_b, out_hbm.at[bv_b], wb).wait()

    return gather_kernel


_gather = _build_gather()


def kernel(table, feedid):
    out_pad = _gather(table.T, feedid)
    return out_pad[:, :DIM]


# speedup vs baseline: 1.0760x; 1.0760x over previous
import functools

import jax
import jax.numpy as jnp
from jax import lax
from jax.experimental import pallas as pl
from jax.experimental.pallas import tpu as pltpu
from jax.experimental.pallas import tpu_sc as plsc

VOCAB = 1000000
DIM = 32
BATCH = 16384

_NUM_CORES = 2
_NUM_SUBCORES = 16
_NW = _NUM_CORES * _NUM_SUBCORES
_TCOLS = 245
_WIN = _TCOLS * 128
_CW = 768
_NCHUNK = 42
_MAX_S = 999296
_NVEC = BATCH // 16


def _build_gather():
    mesh = plsc.VectorSubcoreMesh(core_axis_name="c", subcore_axis_name="s")

    @functools.partial(
        pl.kernel,
        mesh=mesh,
        out_type=jax.ShapeDtypeStruct((BATCH, 128), jnp.float32),
        scratch_types=[
            pltpu.VMEM((BATCH,), jnp.int32),
            pltpu.VMEM((BATCH,), jnp.int32),
            pltpu.VMEM((BATCH,), jnp.int32),
            pltpu.VMEM((DIM, _CW), jnp.float32),
            pltpu.VMEM((DIM, _CW), jnp.float32),
            pltpu.VMEM((DIM, _CW), jnp.float32),
            pltpu.VMEM((16, 128), jnp.float32),
            pltpu.VMEM((16, 128), jnp.float32),
            pltpu.VMEM((16,), jnp.int32),
            pltpu.VMEM((16,), jnp.int32),
            pltpu.SemaphoreType.DMA,
            pltpu.SemaphoreType.DMA,
            pltpu.SemaphoreType.DMA,
            pltpu.SemaphoreType.DMA,
            pltpu.SemaphoreType.DMA,
        ],
        compiler_params=pltpu.CompilerParams(
            needs_layout_passes=False, skip_device_barrier=True
        ),
    )
    def gather_kernel(tbl, idx_hbm, out_hbm, idx_v, dense, chits,
                      ch_a, ch_b, ch_c, rb_a, rb_b, bv_a, bv_b,
                      sa, sb, sc, wa, wb):
        w = lax.axis_index("s") * _NUM_CORES + lax.axis_index("c")
        lo = w * _WIN
        hi = jnp.minimum(lo + _WIN, VOCAB)
        iota = lax.iota(jnp.int32, 16)

        def chunk_start(k):
            return jnp.minimum(lo + k * _CW, _MAX_S)

        def enq_chunk(buf, k, sem):
            coff = pl.multiple_of(chunk_start(k), 128)
            pltpu.async_copy(tbl.at[:, pl.ds(coff, _CW)], buf, sem)

        def drain_chunk(buf, sem):
            pltpu.make_async_copy(tbl.at[:, pl.ds(0, _CW)], buf, sem).wait()

        pltpu.sync_copy(idx_hbm, idx_v)
        enq_chunk(ch_a, 0, sa)
        enq_chunk(ch_b, 1, sb)
        enq_chunk(ch_c, 2, sc)

        def scan_body(g, cur):
            iv = idx_v[pl.ds(pl.multiple_of(g * 16, 16), 16)]
            m = (iv >= lo) & (iv < hi)
            mi = m.astype(jnp.int32)
            cs = plsc.cumsum(mi)
            packed = (iv - lo) * BATCH + g * 16 + iota
            plsc.store_scatter(dense, [cur + cs - mi], packed, mask=m)
            return cur + cs[15]

        m_total = lax.fori_loop(0, _NVEC, scan_body, 0)
        n_dvec = (m_total + 15) >> 4

        def do_group(h0, m_k, s_rel, buf, rb, bv, sem, flag):
            guard = h0 < m_k

            @pl.when(guard)
            def _():
                hv = chits[pl.ds(pl.multiple_of(h0, 16), 16)]
                valid = (h0 + iota) < m_k
                hv2 = jnp.where(valid, hv, jnp.broadcast_to(hv[0], (16,)))
                jc = (hv2 >> 14) - s_rel
                b = hv2 & (BATCH - 1)

                @pl.when(flag > 0)
                def _():
                    pltpu.make_async_copy(rb, out_hbm.at[bv], sem).wait()

                for f in range(DIM):
                    fv = jnp.broadcast_to(jnp.int32(f), (16,))
                    vals = plsc.load_gather(buf, [fv, jc])
                    plsc.store_scatter(rb, [iota, fv], vals)
                bv[...] = b
                pltpu.async_copy(rb, out_hbm.at[bv], sem)

            return jnp.where(guard, jnp.int32(1), flag)

        def process_chunk(k, buf, sem, fa, fb):
            drain_chunk(buf, sem)
            s_rel = chunk_start(k) - lo

            def cc_body(v, cur):
                dv = dense[pl.ds(pl.multiple_of(v * 16, 16), 16)]
                rel = dv >> 14
                mm = ((v * 16 + iota) < m_total) \
                    & (rel >= k * _CW) & (rel < (k + 1) * _CW)
                mi = mm.astype(jnp.int32)
                cs = plsc.cumsum(mi)
                plsc.store_scatter(chits, [cur + cs - mi], dv, mask=mm)
                return cur + cs[15]

            m_k = lax.fori_loop(0, n_dvec, cc_body, 0)

            def ex_body(t, fs):
                f1 = do_group(t * 32, m_k, s_rel, buf, rb_a, bv_a, wa, fs[0])
                f2 = do_group(t * 32 + 16, m_k, s_rel, buf, rb_b, bv_b, wb,
                              fs[1])
                return (f1, f2)

            return lax.fori_loop(0, (m_k + 31) >> 5, ex_body, (fa, fb))

        def chunk_triple(j, fs):
            k = j * 3
            fa, fb = fs

            for off, buf, sem in ((0, ch_a, sa), (1, ch_b, sb), (2, ch_c, sc)):
                fa, fb = process_chunk(k + off, buf, sem, fa, fb)

                @pl.when(k + off + 3 < _NCHUNK)
                def _(buf=buf, sem=sem, off=off):
                    enq_chunk(buf, k + off + 3, sem)

            return (fa, fb)

        fa, fb = lax.fori_loop(0, _NCHUNK // 3, chunk_triple,
                               (jnp.int32(0), jnp.int32(0)))

        @pl.when(fa > 0)
        def _():
            pltpu.make_async_copy(rb_a, out_hbm.at[bv_a], wa).wait()

        @pl.when(fb > 0)
        def _():
            pltpu.make_async_copy(rb_b, out_hbm.at[bv_b], wb).wait()

    return gather_kernel


_gather = _build_gather()


def kernel(table, feedid):
    out_pad = _gather(table.T, feedid)
    return out_pad[:, :DIM]
